# Initial kernel scaffold; baseline (speedup 1.0000x reference)
import jax
import jax.numpy as jnp
from jax.experimental import pallas as pl


def kernel(author_x, paper_x, src_index, dst_index, W_author, b_author, W_paper, b_paper, W_scorer, b_scorer):
    raise NotImplementedError("write your pallas kernel here")


# speedup vs baseline: 108.2432x; 108.2432x over previous
import functools

import jax
import jax.numpy as jnp
from jax import lax
from jax.experimental import pallas as pl
from jax.experimental.pallas import tpu as pltpu
from jax.experimental.pallas import tpu_sc as plsc

_NC = 2
_NS = 16
_NW = _NC * _NS
_L = 16


def _encode_body(ax_ref, px_ref, pa_ref, pp_ref, c_ref, oa_ref, op_ref):
    oa_ref[...] = jnp.dot(ax_ref[...], pa_ref[...],
                          preferred_element_type=jnp.float32)
    op_ref[...] = jnp.dot(px_ref[...], pp_ref[...],
                          preferred_element_type=jnp.float32) + c_ref[0]


def _encode(ax_rs, px_rs, pa, pp, const):
    ra = ax_rs.shape[0]
    rp = px_rs.shape[0]
    g = pa.shape[1]
    return pl.pallas_call(
        _encode_body,
        out_shape=[
            jax.ShapeDtypeStruct((ra, g), jnp.float32),
            jax.ShapeDtypeStruct((rp, g), jnp.float32),
        ],
        in_specs=[
            pl.BlockSpec(memory_space=pltpu.VMEM),
            pl.BlockSpec(memory_space=pltpu.VMEM),
            pl.BlockSpec(memory_space=pltpu.VMEM),
            pl.BlockSpec(memory_space=pltpu.VMEM),
            pl.BlockSpec(memory_space=pltpu.SMEM),
        ],
    )(ax_rs, px_rs, pa, pp, const)


def _make_edge_kernel(e_total, na2, np2, k):
    e_per = e_total // _NW
    chunks = e_per // k
    steps = k // _L
    mesh = plsc.VectorSubcoreMesh(
        core_axis_name="c", subcore_axis_name="s",
        num_cores=_NC, num_subcores=_NS)

    @functools.partial(
        pl.kernel,
        out_type=jax.ShapeDtypeStruct((e_total,), jnp.float32),
        mesh=mesh,
        compiler_params=pltpu.CompilerParams(needs_layout_passes=False),
        scratch_types=[
            pltpu.VMEM((na2,), jnp.int32),
            pltpu.VMEM((np2,), jnp.int32),
            pltpu.VMEM((k,), jnp.int32),
            pltpu.VMEM((k,), jnp.int32),
            pltpu.VMEM((k,), jnp.float32),
        ],
    )
    def edge_kernel(sa_hbm, sp_hbm, src_hbm, dst_hbm, out_hbm,
                    sa_v, sp_v, src_v, dst_v, out_v):
        wid = lax.axis_index("s") * _NC + lax.axis_index("c")
        base = wid * e_per
        pltpu.sync_copy(sa_hbm, sa_v)
        pltpu.sync_copy(sp_hbm, sp_v)

        def chunk(c, carry):
            off = base + c * k
            pltpu.sync_copy(src_hbm.at[pl.ds(off, k)], src_v)
            pltpu.sync_copy(dst_hbm.at[pl.ds(off, k)], dst_v)

            def inner(i, carry2):
                sl = pl.ds(i * _L, _L)
                s = src_v[sl]
                d = dst_v[sl]
                ws = plsc.load_gather(sa_v, [s >> 1])
                wd = plsc.load_gather(sp_v, [d >> 1])
                ss = (s & 1) << 4
                sd = (d & 1) << 4
                fs = plsc.bitcast(lax.shift_right_logical(ws, ss) << 16,
                                  jnp.float32)
                fd = plsc.bitcast(lax.shift_right_logical(wd, sd) << 16,
                                  jnp.float32)
                out_v[sl] = fs + fd
                return carry2

            lax.fori_loop(0, steps, inner, 0, unroll=2)
            pltpu.sync_copy(out_v, out_hbm.at[pl.ds(off, k)])
            return carry

        lax.fori_loop(0, chunks, chunk, 0)

    return edge_kernel


def _pack_bf16(table_f32):
    bf = table_f32.astype(jnp.bfloat16).reshape(-1, 2)
    return lax.bitcast_convert_type(bf, jnp.int32)


def kernel(author_x, paper_x, src_index, dst_index,
           W_author, b_author, W_paper, b_paper, W_scorer, b_scorer):
    d = author_x.shape[1]
    na = author_x.shape[0]
    npp = paper_x.shape[0]
    e = src_index.shape[0]
    g = 128 // d

    w1 = W_scorer[0, :d]
    w2 = W_scorer[0, d:]
    v_a = W_author.T @ w1
    v_p = W_paper.T @ w2
    const = (b_scorer[0] + b_author @ w1 + b_paper @ w2).reshape(1)

    lane = jnp.arange(128)
    grp = jnp.arange(g)
    msk = (lane[:, None] // d) == grp[None, :]
    pa = jnp.where(msk, jnp.tile(v_a, g)[:, None], 0.0).astype(jnp.float32)
    pp = jnp.where(msk, jnp.tile(v_p, g)[:, None], 0.0).astype(jnp.float32)

    ax_rs = author_x.reshape(na * d // 128, 128)
    px_rs = paper_x.reshape(npp * d // 128, 128)

    sa, sp = _encode(ax_rs, px_rs, pa, pp, const)
    sa_packed = _pack_bf16(sa.reshape(na))
    sp_packed = _pack_bf16(sp.reshape(npp))

    k = 4000
    unit = _NW * k
    e_pad = ((e + unit - 1) // unit) * unit
    src = src_index.astype(jnp.int32)
    dst = dst_index.astype(jnp.int32)
    if e_pad != e:
        pad = e_pad - e
        src = jnp.concatenate([src, jnp.zeros((pad,), jnp.int32)])
        dst = jnp.concatenate([dst, jnp.zeros((pad,), jnp.int32)])

    edge_kernel = _make_edge_kernel(e_pad, sa_packed.shape[0],
                                    sp_packed.shape[0], k)
    out = edge_kernel(sa_packed, sp_packed, src, dst)
    return out[:e] if e_pad != e else out
